# baseline (speedup 1.0000x reference)
import functools

import jax
import jax.numpy as jnp
from jax import lax
from jax.experimental import pallas as pl
from jax.experimental.pallas import tpu as pltpu
from jax.experimental.pallas import tpu_sc as plsc

N, E, F, H1, H2, C = 10000, 320000, 128, 100, 100, 16
DP = 128
NPAD = 10240
NC, NS = 2, 16
CHUNK = 72
CPT = 144
EROWS = NC * NS * CPT
EPAD = EROWS * CHUNK
ROWS_PER_TILE = NPAD // NS
NBUF = 4
SUPB = 48
MBLK = 1000


def _make_agg():
    def body(support, src2d, dst2d, zeros, out,
             src_all, dst_all, agg, *rows_and_sems):
        rows = rows_and_sems[:NBUF]
        sems = rows_and_sems[NBUF:2 * NBUF]
        ssems = rows_and_sems[2 * NBUF:]
        c = lax.axis_index("c")
        s = lax.axis_index("s")
        t = c * NS + s
        slab = pl.ds(s * ROWS_PER_TILE, ROWS_PER_TILE)
        pltpu.sync_copy(zeros, agg.at[slab])
        plsc.subcore_barrier()

        def gwait(b):
            pltpu.make_async_copy(
                support.at[pl.ds(0, CHUNK)], rows[b], sems[b]).wait()

        def swait(b):
            pltpu.make_async_copy(
                rows[b], agg.at[pl.ds(0, CHUNK)], ssems[b]).wait()

        def gfire(j, b):
            pltpu.async_copy(support.at[src_all.at[j]], rows[b], sems[b])

        def sfire(j, b):
            pltpu.async_copy(rows[b], agg.at[dst_all.at[j]], ssems[b],
                             add=True)

        for half in range(CPT // SUPB):
            base = t * CPT + half * SUPB
            pltpu.sync_copy(src2d.at[pl.ds(base, SUPB)], src_all)
            pltpu.sync_copy(dst2d.at[pl.ds(base, SUPB)], dst_all)
            gfire(0, 0)
            gfire(1, 1)
            gwait(0); sfire(0, 0); gfire(2, 2)
            gwait(1); sfire(1, 1); gfire(3, 3)
            gwait(2); sfire(2, 2); swait(0); gfire(4, 0)
            gwait(3); sfire(3, 3); swait(1); gfire(5, 1)

            def step(g, carry):
                for b in range(NBUF):
                    k = g * NBUF + b
                    gwait(b)
                    sfire(k, b)
                    b2 = (b + 2) % NBUF
                    swait(b2)
                    jn = jnp.where(k + 2 < SUPB, k + 2, b2)
                    gfire(jn, b2)
                return carry

            lax.fori_loop(1, SUPB // NBUF, step, 0)
            gwait(0)
            gwait(1)
            swait(2)
            swait(3)
        plsc.subcore_barrier()
        pltpu.sync_copy(agg.at[slab], out.at[c, slab])

    return pl.kernel(
        body,
        out_type=jax.ShapeDtypeStruct((NC, NPAD, DP), jnp.float32),
        mesh=plsc.VectorSubcoreMesh(core_axis_name="c", subcore_axis_name="s"),
        scratch_types=(
            [pltpu.VMEM((SUPB, CHUNK), jnp.int32),
             pltpu.VMEM((SUPB, CHUNK), jnp.int32),
             pltpu.VMEM_SHARED((NPAD, DP), jnp.float32)]
            + [pltpu.VMEM((CHUNK, DP), jnp.float32)] * NBUF
            + [pltpu.SemaphoreType.DMA] * (2 * NBUF)
        ),
    )


_aggregate = _make_agg()


def _mm_body(x_ref, w_ref, o_ref):
    o_ref[...] = jnp.dot(x_ref[...], w_ref[...],
                         preferred_element_type=jnp.float32)


def _combine_mm_body(p_ref, b_ref, w_ref, o_ref):
    h = jnp.maximum(p_ref[0] + p_ref[1] + b_ref[...], 0.0)
    o_ref[...] = jnp.dot(h, w_ref[...], preferred_element_type=jnp.float32)


def _head_body(p_ref, b2_ref, w3_ref, b3_ref, o_ref):
    h = jnp.maximum(p_ref[0] + p_ref[1] + b2_ref[...], 0.0)
    logits = jnp.dot(h, w3_ref[...], preferred_element_type=jnp.float32)
    logits = logits + b3_ref[...]
    m = jnp.max(logits, axis=1, keepdims=True)
    lse = jnp.log(jnp.sum(jnp.exp(logits - m), axis=1, keepdims=True)) + m
    o_ref[...] = logits - lse


def _mm(x, w):
    grid = N // MBLK
    return pl.pallas_call(
        _mm_body,
        grid=(grid,),
        in_specs=[
            pl.BlockSpec((MBLK, F), lambda i: (i, 0)),
            pl.BlockSpec((F, DP), lambda i: (0, 0)),
        ],
        out_specs=pl.BlockSpec((MBLK, DP), lambda i: (i, 0)),
        out_shape=jax.ShapeDtypeStruct((N, DP), jnp.float32),
    )(x, w)


def _combine_mm(parts, b, w):
    grid = N // MBLK
    return pl.pallas_call(
        _combine_mm_body,
        grid=(grid,),
        in_specs=[
            pl.BlockSpec((NC, MBLK, DP), lambda i: (0, i, 0)),
            pl.BlockSpec((1, DP), lambda i: (0, 0)),
            pl.BlockSpec((DP, DP), lambda i: (0, 0)),
        ],
        out_specs=pl.BlockSpec((MBLK, DP), lambda i: (i, 0)),
        out_shape=jax.ShapeDtypeStruct((N, DP), jnp.float32),
    )(parts, b, w)


def _head(parts, b2, w3, b3):
    grid = N // MBLK
    return pl.pallas_call(
        _head_body,
        grid=(grid,),
        in_specs=[
            pl.BlockSpec((NC, MBLK, DP), lambda i: (0, i, 0)),
            pl.BlockSpec((1, DP), lambda i: (0, 0)),
            pl.BlockSpec((DP, C), lambda i: (0, 0)),
            pl.BlockSpec((1, C), lambda i: (0, 0)),
        ],
        out_specs=pl.BlockSpec((MBLK, C), lambda i: (i, 0)),
        out_shape=jax.ShapeDtypeStruct((N, C), jnp.float32),
    )(parts, b2, w3, b3)


def kernel(x, edge_index, W1, b1, W2, b2, W3, b3):
    src = edge_index[0]
    dst = edge_index[1]

    w1p = jnp.zeros((F, DP), jnp.float32).at[:, :H1].set(W1)
    b1p = jnp.zeros((1, DP), jnp.float32).at[0, :H1].set(b1)
    w2p = jnp.zeros((DP, DP), jnp.float32).at[:H1, :H2].set(W2)
    b2p = jnp.zeros((1, DP), jnp.float32).at[0, :H2].set(b2)
    w3p = jnp.zeros((DP, C), jnp.float32).at[:H2, :].set(W3)
    b3p = b3.reshape(1, C)

    pidx = jnp.arange(EPAD - E, dtype=jnp.int32)
    srcp = jnp.concatenate([src, pidx % N]).reshape(EROWS, CHUNK)
    dstp = jnp.concatenate(
        [dst, N + pidx % (NPAD - N)]).reshape(EROWS, CHUNK)
    del pidx
    zeros = jnp.zeros((ROWS_PER_TILE, DP), jnp.float32)

    support1 = _mm(x, w1p)
    parts1 = _aggregate(support1, srcp, dstp, zeros)
    support2 = _combine_mm(parts1, b1p, w2p)
    parts2 = _aggregate(support2, srcp, dstp, zeros)
    return _head(parts2, b2p, w3p, b3p)


# speedup vs baseline: 1.1076x; 1.1076x over previous
import functools

import jax
import jax.numpy as jnp
from jax import lax
from jax.experimental import pallas as pl
from jax.experimental.pallas import tpu as pltpu
from jax.experimental.pallas import tpu_sc as plsc

N, E, F, H1, H2, C = 10000, 320000, 128, 100, 100, 16
DP = 128
NPAD = 10240
NC, NS = 2, 16
CHUNK = 88
CPT = 120
EROWS = NC * NS * CPT
EPAD = EROWS * CHUNK
ROWS_PER_TILE = NPAD // NS
NBUF = 3
SUPB = 40
MBLK = 1000


def _make_agg():
    def body(support, src2d, dst2d, zeros, out,
             src_all, dst_all, agg, *rows_and_sems):
        rows = rows_and_sems[:NBUF]
        sems = rows_and_sems[NBUF:2 * NBUF]
        ssems = rows_and_sems[2 * NBUF:]
        c = lax.axis_index("c")
        s = lax.axis_index("s")
        t = c * NS + s
        slab = pl.ds(s * ROWS_PER_TILE, ROWS_PER_TILE)
        pltpu.sync_copy(zeros, agg.at[slab])
        plsc.subcore_barrier()

        def gwait(b):
            pltpu.make_async_copy(
                support.at[pl.ds(0, CHUNK)], rows[b], sems[b]).wait()

        def swait(b):
            pltpu.make_async_copy(
                rows[b], agg.at[pl.ds(0, CHUNK)], ssems[b]).wait()

        def gfire(j, b):
            pltpu.async_copy(support.at[src_all.at[j]], rows[b], sems[b])

        def sfire(j, b):
            pltpu.async_copy(rows[b], agg.at[dst_all.at[j]], ssems[b],
                             add=True)

        for half in range(CPT // SUPB):
            base = t * CPT + half * SUPB
            pltpu.sync_copy(src2d.at[pl.ds(base, SUPB)], src_all)
            pltpu.sync_copy(dst2d.at[pl.ds(base, SUPB)], dst_all)
            gfire(0, 0)
            gfire(1, 1)
            gwait(0); sfire(0, 0); gfire(2, 2)
            gwait(1); sfire(1, 1); swait(0); gfire(3, 0)
            gwait(2); sfire(2, 2); swait(1); gfire(4, 1)

            def step(g, carry):
                for b in range(NBUF):
                    k = g * NBUF + b
                    gwait(b)
                    sfire(k, b)
                    b2 = (b + 2) % NBUF
                    swait(b2)
                    jn = jnp.where(k + 2 < SUPB, k + 2, b2)
                    gfire(jn, b2)
                return carry

            lax.fori_loop(1, 13, step, 0)
            gwait(0); sfire(39, 0); swait(2); gfire(2, 2)
            gwait(1)
            gwait(2)
            swait(0)
        plsc.subcore_barrier()
        pltpu.sync_copy(agg.at[slab], out.at[c, slab])

    return pl.kernel(
        body,
        out_type=jax.ShapeDtypeStruct((NC, NPAD, DP), jnp.float32),
        mesh=plsc.VectorSubcoreMesh(core_axis_name="c", subcore_axis_name="s"),
        scratch_types=(
            [pltpu.VMEM((SUPB, CHUNK), jnp.int32),
             pltpu.VMEM((SUPB, CHUNK), jnp.int32),
             pltpu.VMEM_SHARED((NPAD, DP), jnp.float32)]
            + [pltpu.VMEM((CHUNK, DP), jnp.float32)] * NBUF
            + [pltpu.SemaphoreType.DMA] * (2 * NBUF)
        ),
    )


_aggregate = _make_agg()


def _mm_body(x_ref, w_ref, o_ref):
    o_ref[...] = jnp.dot(x_ref[...], w_ref[...],
                         preferred_element_type=jnp.float32)


def _combine_mm_body(p_ref, b_ref, w_ref, o_ref):
    h = jnp.maximum(p_ref[0] + p_ref[1] + b_ref[...], 0.0)
    o_ref[...] = jnp.dot(h, w_ref[...], preferred_element_type=jnp.float32)


def _head_body(p_ref, b2_ref, w3_ref, b3_ref, o_ref):
    h = jnp.maximum(p_ref[0] + p_ref[1] + b2_ref[...], 0.0)
    logits = jnp.dot(h, w3_ref[...], preferred_element_type=jnp.float32)
    logits = logits + b3_ref[...]
    m = jnp.max(logits, axis=1, keepdims=True)
    lse = jnp.log(jnp.sum(jnp.exp(logits - m), axis=1, keepdims=True)) + m
    o_ref[...] = logits - lse


def _mm(x, w):
    grid = N // MBLK
    return pl.pallas_call(
        _mm_body,
        grid=(grid,),
        in_specs=[
            pl.BlockSpec((MBLK, F), lambda i: (i, 0)),
            pl.BlockSpec((F, DP), lambda i: (0, 0)),
        ],
        out_specs=pl.BlockSpec((MBLK, DP), lambda i: (i, 0)),
        out_shape=jax.ShapeDtypeStruct((N, DP), jnp.float32),
    )(x, w)


def _combine_mm(parts, b, w):
    grid = N // MBLK
    return pl.pallas_call(
        _combine_mm_body,
        grid=(grid,),
        in_specs=[
            pl.BlockSpec((NC, MBLK, DP), lambda i: (0, i, 0)),
            pl.BlockSpec((1, DP), lambda i: (0, 0)),
            pl.BlockSpec((DP, DP), lambda i: (0, 0)),
        ],
        out_specs=pl.BlockSpec((MBLK, DP), lambda i: (i, 0)),
        out_shape=jax.ShapeDtypeStruct((N, DP), jnp.float32),
    )(parts, b, w)


def _head(parts, b2, w3, b3):
    grid = N // MBLK
    return pl.pallas_call(
        _head_body,
        grid=(grid,),
        in_specs=[
            pl.BlockSpec((NC, MBLK, DP), lambda i: (0, i, 0)),
            pl.BlockSpec((1, DP), lambda i: (0, 0)),
            pl.BlockSpec((DP, C), lambda i: (0, 0)),
            pl.BlockSpec((1, C), lambda i: (0, 0)),
        ],
        out_specs=pl.BlockSpec((MBLK, C), lambda i: (i, 0)),
        out_shape=jax.ShapeDtypeStruct((N, C), jnp.float32),
    )(parts, b2, w3, b3)


def kernel(x, edge_index, W1, b1, W2, b2, W3, b3):
    src = edge_index[0]
    dst = edge_index[1]

    w1p = jnp.zeros((F, DP), jnp.float32).at[:, :H1].set(W1)
    b1p = jnp.zeros((1, DP), jnp.float32).at[0, :H1].set(b1)
    w2p = jnp.zeros((DP, DP), jnp.float32).at[:H1, :H2].set(W2)
    b2p = jnp.zeros((1, DP), jnp.float32).at[0, :H2].set(b2)
    w3p = jnp.zeros((DP, C), jnp.float32).at[:H2, :].set(W3)
    b3p = b3.reshape(1, C)

    pidx = jnp.arange(EPAD - E, dtype=jnp.int32)
    srcp = jnp.concatenate([src, pidx % N]).reshape(EROWS, CHUNK)
    dstp = jnp.concatenate(
        [dst, N + pidx % (NPAD - N)]).reshape(EROWS, CHUNK)
    del pidx
    zeros = jnp.zeros((ROWS_PER_TILE, DP), jnp.float32)

    support1 = _mm(x, w1p)
    parts1 = _aggregate(support1, srcp, dstp, zeros)
    support2 = _combine_mm(parts1, b1p, w2p)
    parts2 = _aggregate(support2, srcp, dstp, zeros)
    return _head(parts2, b2p, w3p, b3p)
